# baseline (speedup 1.0000x reference)
import functools

import jax
import jax.numpy as jnp
from jax.experimental import pallas as pl


def _ecc_block_kernel(x_ref, proto_ref, out_ref, *, num_classes):
    xb = x_ref[0]
    proto = proto_ref[...]
    p_sq = jnp.sum(proto * proto, axis=1)[:, None, None]
    dots = jax.lax.dot_general(
        proto, xb, (((1,), (0,)), ((), ())),
        preferred_element_type=jnp.float32)
    sq = p_sq - 2.0 * dots
    kp, hb, w = sq.shape
    sqm = jnp.max(sq.reshape(num_classes, kp // num_classes, hb, w), axis=1)
    x_sq = jnp.sum(xb * xb, axis=0, keepdims=True)
    out_ref[0] = jnp.sqrt(jnp.maximum(sqm + x_sq, 0.0))


def kernel(x, gt, prototype):
    del gt
    B, C, H, W = x.shape
    K, P, _ = prototype.shape
    KP = K * P
    HB = 24

    proto = prototype.reshape(KP, C)

    return pl.pallas_call(
        functools.partial(_ecc_block_kernel, num_classes=K),
        grid=(B, H // HB),
        in_specs=[
            pl.BlockSpec((1, C, HB, W), lambda b, h: (b, 0, h, 0)),
            pl.BlockSpec((KP, C), lambda b, h: (0, 0)),
        ],
        out_specs=pl.BlockSpec((1, K, HB, W), lambda b, h: (b, 0, h, 0)),
        out_shape=jax.ShapeDtypeStruct((B, K, H, W), jnp.float32),
    )(x, proto)
